# Initial kernel scaffold; baseline (speedup 1.0000x reference)
import jax
import jax.numpy as jnp
from jax.experimental import pallas as pl


def kernel(x_nchw, w1_oihw, b1, w2_oihw, b2):
    raise NotImplementedError("write your pallas kernel here")


# speedup vs baseline: 1.0486x; 1.0486x over previous
import functools

import jax
import jax.numpy as jnp
from jax import lax
from jax.experimental import pallas as pl
from jax.experimental.pallas import tpu as pltpu

_P = 128


def _conv_block_kernel(x_ref, w1_ref, b1_ref, w2_ref, b2_ref, out_ref,
                       xpad_ref, patch_ref, *, H, W, Cin, Cout, bb):
    HW = H * W

    xpad_ref[...] = jnp.zeros(xpad_ref.shape, xpad_ref.dtype)

    w1 = w1_ref[...]
    w2 = w2_ref[...]
    b1 = b1_ref[...]
    b2 = b2_ref[...]

    col = lax.broadcasted_iota(jnp.int32, (1, HW), 1) % W
    not_first_col = col > 0
    not_last_col = col < (W - 1)

    taps = [(dy, dx) for dy in (-1, 0, 1) for dx in (-1, 0, 1)]

    def build_patch(masked):
        for t, (dy, dx) in enumerate(taps):
            s = dy * W + dx
            piece = xpad_ref[:, _P + s:_P + s + HW]
            if masked and dx == -1:
                piece = jnp.where(not_first_col, piece, 0)
            elif masked and dx == 1:
                piece = jnp.where(not_last_col, piece, 0)
            patch_ref[t * Cin:(t + 1) * Cin, :] = piece
        return patch_ref[...]

    for b in range(bb):
        xb = x_ref[b]
        xpad_ref[:, _P:_P + HW] = xb.astype(jnp.bfloat16)

        p1 = build_patch(masked=True)
        mid = jnp.dot(w1, p1, preferred_element_type=jnp.float32) + b1 + xb
        mid = jnp.maximum(mid, 0.0)

        xpad_ref[:, _P:_P + HW] = mid.astype(jnp.bfloat16)
        p2 = build_patch(masked=False)
        out = jnp.dot(w2, p2, preferred_element_type=jnp.float32) + b2
        out_ref[b] = jnp.maximum(out, 0.0).astype(out_ref.dtype)


@jax.jit
def _cnn_conv_block(x_nchw, w1_oihw, b1, w2_oihw, b2):
    N, Cin, H, W = x_nchw.shape
    Cout = w2_oihw.shape[0]
    HW = H * W
    HWP = HW + 2 * _P
    bb = 2 if N % 2 == 0 else 1

    x_flat = x_nchw.reshape(N, Cin, HW)
    w1s = jnp.transpose(w1_oihw, (0, 2, 3, 1)).reshape(Cin, 9 * Cin)
    w2s = jnp.transpose(w2_oihw, (0, 2, 3, 1)).reshape(Cout, 9 * Cin)
    w1s = w1s.astype(jnp.bfloat16)
    w2s = w2s.astype(jnp.bfloat16)
    b1c = b1.reshape(Cin, 1)
    b2c = b2.reshape(Cout, 1)

    _conv_fn = functools.partial(_conv_block_kernel, H=H, W=W, Cin=Cin,
                                 Cout=Cout, bb=bb)

    flops = 2 * 9 * Cin * (Cin + Cout) * HW * N
    bytes_accessed = (x_flat.size + N * Cout * HW) * x_nchw.dtype.itemsize \
        + (w1s.size + w2s.size) * 2 + (b1c.size + b2c.size) * 4

    out_full = pl.pallas_call(
        _conv_fn,
        out_shape=jax.ShapeDtypeStruct((N, Cout, HW), x_nchw.dtype),
        grid=(N // bb,),
        in_specs=[
            pl.BlockSpec((bb, Cin, HW), lambda n: (n, 0, 0)),
            pl.BlockSpec((Cin, 9 * Cin), lambda n: (0, 0)),
            pl.BlockSpec((Cin, 1), lambda n: (0, 0)),
            pl.BlockSpec((Cout, 9 * Cin), lambda n: (0, 0)),
            pl.BlockSpec((Cout, 1), lambda n: (0, 0)),
        ],
        out_specs=pl.BlockSpec((bb, Cout, HW), lambda n: (n, 0, 0)),
        scratch_shapes=[pltpu.VMEM((Cin, HWP), jnp.bfloat16),
                        pltpu.VMEM((9 * Cin, HW), jnp.bfloat16)],
        compiler_params=pltpu.CompilerParams(
            dimension_semantics=("parallel",)),
        cost_estimate=pl.CostEstimate(flops=flops, transcendentals=0,
                                      bytes_accessed=bytes_accessed),
    )(x_flat, w1s, b1c, w2s, b2c)

    return out_full.reshape(N, Cout, H, W)[:, :, 1:H - 1, 1:W - 1]


def kernel(x_nchw, w1_oihw, b1, w2_oihw, b2):
    return _cnn_conv_block(x_nchw, w1_oihw, b1, w2_oihw, b2)


# speedup vs baseline: 1.5996x; 1.5255x over previous
import functools

import jax
import jax.numpy as jnp
from jax import lax
from jax.experimental import pallas as pl
from jax.experimental.pallas import tpu as pltpu

_P = 128


def _conv_block_kernel(x_ref, w1_ref, b1_ref, w2_ref, b2_ref, out_ref,
                       xpad_ref, patch_ref, *, H, W, Cin, Cout, bb):
    HW = H * W

    xpad_ref[...] = jnp.zeros(xpad_ref.shape, xpad_ref.dtype)

    w1 = w1_ref[...]
    w2 = w2_ref[...]
    b1 = b1_ref[...]
    b2 = b2_ref[...]

    col = lax.broadcasted_iota(jnp.int32, (1, HW), 1) % W
    not_first_col = col > 0
    not_last_col = col < (W - 1)

    taps = [(dy, dx) for dy in (-1, 0, 1) for dx in (-1, 0, 1)]

    def build_patch(masked):
        for t, (dy, dx) in enumerate(taps):
            s = dy * W + dx
            piece = xpad_ref[:, _P + s:_P + s + HW]
            if masked and dx == -1:
                piece = jnp.where(not_first_col, piece, 0)
            elif masked and dx == 1:
                piece = jnp.where(not_last_col, piece, 0)
            patch_ref[t * Cin:(t + 1) * Cin, :] = piece
        return patch_ref[...]

    for b in range(bb):
        xb = x_ref[b].astype(jnp.bfloat16).reshape(Cin, HW)
        xpad_ref[:, _P:_P + HW] = xb

        p1 = build_patch(masked=True)
        mid = jnp.dot(w1, p1, preferred_element_type=jnp.float32) + b1 \
            + xb.astype(jnp.float32)
        mid = jnp.maximum(mid, 0.0)

        xpad_ref[:, _P:_P + HW] = mid.astype(jnp.bfloat16)
        p2 = build_patch(masked=False)
        out = jnp.dot(w2, p2, preferred_element_type=jnp.float32) + b2
        out = jnp.maximum(out, 0.0)
        out_ref[b] = out.reshape(Cout, H, W)[:, 1:H - 1, 1:W - 1]


@jax.jit
def _cnn_conv_block(x_nchw, w1_oihw, b1, w2_oihw, b2):
    N, Cin, H, W = x_nchw.shape
    Cout = w2_oihw.shape[0]
    HW = H * W
    HWP = HW + 2 * _P
    bb = 2 if N % 2 == 0 else 1

    w1s = jnp.transpose(w1_oihw, (0, 2, 3, 1)).reshape(Cin, 9 * Cin)
    w2s = jnp.transpose(w2_oihw, (0, 2, 3, 1)).reshape(Cout, 9 * Cin)
    w1s = w1s.astype(jnp.bfloat16)
    w2s = w2s.astype(jnp.bfloat16)
    b1c = b1.reshape(Cin, 1)
    b2c = b2.reshape(Cout, 1)

    _conv_fn = functools.partial(_conv_block_kernel, H=H, W=W, Cin=Cin,
                                 Cout=Cout, bb=bb)

    flops = 2 * 9 * Cin * (Cin + Cout) * HW * N
    bytes_accessed = (x_nchw.size + N * Cout * (H - 2) * (W - 2)) \
        * x_nchw.dtype.itemsize \
        + (w1s.size + w2s.size) * 2 + (b1c.size + b2c.size) * 4

    out = pl.pallas_call(
        _conv_fn,
        out_shape=jax.ShapeDtypeStruct((N, Cout, H - 2, W - 2), x_nchw.dtype),
        grid=(N // bb,),
        in_specs=[
            pl.BlockSpec((bb, Cin, H, W), lambda n: (n, 0, 0, 0)),
            pl.BlockSpec((Cin, 9 * Cin), lambda n: (0, 0)),
            pl.BlockSpec((Cin, 1), lambda n: (0, 0)),
            pl.BlockSpec((Cout, 9 * Cin), lambda n: (0, 0)),
            pl.BlockSpec((Cout, 1), lambda n: (0, 0)),
        ],
        out_specs=pl.BlockSpec((bb, Cout, H - 2, W - 2),
                               lambda n: (n, 0, 0, 0)),
        scratch_shapes=[pltpu.VMEM((Cin, HWP), jnp.bfloat16),
                        pltpu.VMEM((9 * Cin, HW), jnp.bfloat16)],
        compiler_params=pltpu.CompilerParams(
            dimension_semantics=("parallel",)),
        cost_estimate=pl.CostEstimate(flops=flops, transcendentals=0,
                                      bytes_accessed=bytes_accessed),
    )(x_nchw, w1s, b1c, w2s, b2c)

    return out


def kernel(x_nchw, w1_oihw, b1, w2_oihw, b2):
    return _cnn_conv_block(x_nchw, w1_oihw, b1, w2_oihw, b2)
